# baseline (speedup 1.0000x reference)
import functools

import numpy as np
import jax
import jax.numpy as jnp
from jax import lax
from jax.experimental import pallas as pl
from jax.experimental.pallas import tpu as pltpu
from jax.experimental.pallas import tpu_sc as plsc

VOCAB_SIZE = 1000000
D_DIM = 64
BATCH_N = 4096
SEQ_L = 200


def _pe_table() -> np.ndarray:
    pos = np.arange(SEQ_L)[:, np.newaxis].astype(np.float64)
    i = np.arange(D_DIM)[np.newaxis, :].astype(np.float64)
    angle_rates = 1.0 / np.power(10000, 2 * (i // 2) / np.float32(D_DIM))
    angle_rads = pos * angle_rates
    angle_rads[:, 0::2] = np.sin(angle_rads[:, 0::2])
    angle_rads[:, 1::2] = np.cos(angle_rads[:, 1::2])
    return np.asarray(angle_rads, dtype=np.float32)


_PE_T_CONST = np.ascontiguousarray(_pe_table().T)

_INFO = plsc.get_sparse_core_info()
_NC, _NS = _INFO.num_cores, _INFO.num_subcores
NW = _NC * _NS

SEQ_PER_W = BATCH_N // NW
_SUBS = [(0, 128), (128, 72)]
LANES = 16
_L0S = [16 * i for i in range(12)] + [SEQ_L - LANES]


def _sc_embed(table2, idx2d, pe_t):
    mesh = plsc.VectorSubcoreMesh(core_axis_name="c", subcore_axis_name="s")

    @functools.partial(
        pl.kernel,
        mesh=mesh,
        out_type=jax.ShapeDtypeStruct((BATCH_N, D_DIM, SEQ_L), jnp.float32),
        scratch_types=[
            pltpu.VMEM((SEQ_L,), jnp.int32),
            pltpu.VMEM((SEQ_L,), jnp.int32),
            pltpu.VMEM((SEQ_L, 2 * D_DIM), jnp.float32),
            pltpu.VMEM((SEQ_L, 2 * D_DIM), jnp.float32),
            pltpu.VMEM((D_DIM, SEQ_L), jnp.float32),
            pltpu.VMEM((D_DIM, SEQ_L), jnp.float32),
            pltpu.VMEM((D_DIM, SEQ_L), jnp.float32),
            pltpu.SemaphoreType.DMA,
            pltpu.SemaphoreType.DMA,
            pltpu.SemaphoreType.DMA,
            pltpu.SemaphoreType.DMA,
        ],
        compiler_params=pltpu.CompilerParams(
            use_tc_tiling_on_sc=True, needs_layout_passes=False),
    )
    def body(table_hbm, idx_hbm, pe_hbm, out_hbm,
             idx_a, idx_b, rows_a, rows_b, out_a, out_b, pe_v,
             gsem_a, gsem_b, osem_a, osem_b):
        wid = lax.axis_index("s") * _NC + lax.axis_index("c")
        seq_base = wid * SEQ_PER_W
        pltpu.sync_copy(pe_hbm, pe_v)
        iota = lax.iota(jnp.int32, LANES)

        def fire_gathers(seq, idx_v, rows_v, gsem):
            pltpu.sync_copy(idx_hbm.at[seq], idx_v)
            for off, ln in _SUBS:
                pltpu.async_copy(
                    table_hbm.at[idx_v.at[pl.ds(off, ln)]],
                    rows_v.at[pl.ds(off, ln)],
                    gsem,
                )

        def wait_gathers(idx_v, rows_v, gsem):
            for off, ln in _SUBS:
                pltpu.make_async_copy(
                    table_hbm.at[idx_v.at[pl.ds(off, ln)]],
                    rows_v.at[pl.ds(off, ln)],
                    gsem,
                ).wait()

        def add_transpose(rows_v, out_v):
            def d_body(d, c2):
                col = lax.broadcast(d, (LANES,))
                for l0 in _L0S:
                    sl = pl.ds(l0, LANES)
                    g = plsc.load_gather(rows_v, [iota + l0, col])
                    out_v[d, sl] = g + pe_v[d, sl]
                return c2

            lax.fori_loop(0, D_DIM, d_body, 0)

        def fire_out(seq, out_v, osem):
            pltpu.async_copy(out_v, out_hbm.at[seq], osem)

        def wait_out(seq, out_v, osem):
            pltpu.make_async_copy(out_v, out_hbm.at[seq], osem).wait()

        fire_gathers(seq_base, idx_a, rows_a, gsem_a)
        fire_gathers(seq_base + 1, idx_b, rows_b, gsem_b)
        wait_gathers(idx_a, rows_a, gsem_a)
        add_transpose(rows_a, out_a)
        fire_out(seq_base, out_a, osem_a)
        fire_gathers(seq_base + 2, idx_a, rows_a, gsem_a)
        wait_gathers(idx_b, rows_b, gsem_b)
        add_transpose(rows_b, out_b)
        fire_out(seq_base + 1, out_b, osem_b)
        fire_gathers(seq_base + 3, idx_b, rows_b, gsem_b)

        last = SEQ_PER_W - 1

        def pair_body(p, carry):
            s0 = seq_base + 2 * p
            wait_gathers(idx_a, rows_a, gsem_a)
            wait_out(s0 - 2, out_a, osem_a)
            add_transpose(rows_a, out_a)
            fire_out(s0, out_a, osem_a)
            fire_gathers(seq_base + jnp.minimum(2 * p + 2, last),
                         idx_a, rows_a, gsem_a)
            wait_gathers(idx_b, rows_b, gsem_b)
            wait_out(s0 - 1, out_b, osem_b)
            add_transpose(rows_b, out_b)
            fire_out(s0 + 1, out_b, osem_b)
            fire_gathers(seq_base + jnp.minimum(2 * p + 3, last),
                         idx_b, rows_b, gsem_b)
            return carry

        lax.fori_loop(1, SEQ_PER_W // 2, pair_body, 0)

        wait_gathers(idx_a, rows_a, gsem_a)
        wait_gathers(idx_b, rows_b, gsem_b)
        wait_out(seq_base + SEQ_PER_W - 2, out_a, osem_a)
        wait_out(seq_base + SEQ_PER_W - 1, out_b, osem_b)

    return body(table2, idx2d, pe_t)


def kernel(inputs, table):
    table2 = jnp.pad(table, ((0, 0), (0, D_DIM)))
    pe_t = jnp.asarray(_PE_T_CONST)
    out = _sc_embed(table2, inputs, pe_t)
    return jnp.transpose(out, (0, 2, 1))


# speedup vs baseline: 1.1740x; 1.1740x over previous
import functools

import numpy as np
import jax
import jax.numpy as jnp
from jax import lax
from jax.experimental import pallas as pl
from jax.experimental.pallas import tpu as pltpu
from jax.experimental.pallas import tpu_sc as plsc

VOCAB_SIZE = 1000000
D_DIM = 64
BATCH_N = 4096
SEQ_L = 200


def _pe_table() -> np.ndarray:
    pos = np.arange(SEQ_L)[:, np.newaxis].astype(np.float64)
    i = np.arange(D_DIM)[np.newaxis, :].astype(np.float64)
    angle_rates = 1.0 / np.power(10000, 2 * (i // 2) / np.float32(D_DIM))
    angle_rads = pos * angle_rates
    angle_rads[:, 0::2] = np.sin(angle_rads[:, 0::2])
    angle_rads[:, 1::2] = np.cos(angle_rads[:, 1::2])
    return np.asarray(angle_rads, dtype=np.float32)


_PE_CONST = _pe_table()

_INFO = plsc.get_sparse_core_info()
_NC, _NS = _INFO.num_cores, _INFO.num_subcores
NW = _NC * _NS

B_PER_W = BATCH_N // NW
LANES = 16
VECS_PER_ROW = D_DIM // LANES
BVECS = B_PER_W // LANES


def _sc_embed(table2, idx_t, pe):
    mesh = plsc.VectorSubcoreMesh(core_axis_name="c", subcore_axis_name="s")

    @functools.partial(
        pl.kernel,
        mesh=mesh,
        out_type=jax.ShapeDtypeStruct((SEQ_L, D_DIM, BATCH_N), jnp.float32),
        scratch_types=[
            pltpu.VMEM((SEQ_L, B_PER_W), jnp.int32),
            pltpu.VMEM((B_PER_W, 2 * D_DIM), jnp.float32),
            pltpu.VMEM((B_PER_W, 2 * D_DIM), jnp.float32),
            pltpu.VMEM((D_DIM, B_PER_W), jnp.float32),
            pltpu.VMEM((D_DIM, B_PER_W), jnp.float32),
            pltpu.VMEM((SEQ_L, D_DIM), jnp.float32),
            pltpu.SemaphoreType.DMA,
            pltpu.SemaphoreType.DMA,
            pltpu.SemaphoreType.DMA,
            pltpu.SemaphoreType.DMA,
        ],
        compiler_params=pltpu.CompilerParams(
            use_tc_tiling_on_sc=True, needs_layout_passes=False),
    )
    def body(table_hbm, idx_hbm, pe_hbm, out_hbm,
             idx_v, rows_a, rows_b, out_a, out_b, pe_v,
             gsem_a, gsem_b, osem_a, osem_b):
        wid = lax.axis_index("s") * _NC + lax.axis_index("c")
        b0 = pl.multiple_of(wid * B_PER_W, B_PER_W)
        pltpu.sync_copy(pe_hbm, pe_v)
        pltpu.sync_copy(idx_hbm.at[:, pl.ds(b0, B_PER_W)], idx_v)
        iota = lax.iota(jnp.int32, LANES)

        def fire_gather(l, rows_v, gsem):
            pltpu.async_copy(table_hbm.at[idx_v.at[l]], rows_v, gsem)

        def wait_gather(l, rows_v, gsem):
            pltpu.make_async_copy(
                table_hbm.at[idx_v.at[l]], rows_v, gsem).wait()

        def process(l, rows_v, out_v):
            pevs = [pe_v[l, pl.ds(j * LANES, LANES)]
                    for j in range(VECS_PER_ROW)]

            def add_body(bb, c2):
                for j in range(VECS_PER_ROW):
                    sl = pl.ds(j * LANES, LANES)
                    rows_v[bb, sl] = rows_v[bb, sl] + pevs[j]
                return c2

            lax.fori_loop(0, B_PER_W, add_body, 0)

            def d_body(d, c2):
                col = lax.broadcast(d, (LANES,))
                for j8 in range(BVECS):
                    g = plsc.load_gather(rows_v, [iota + j8 * LANES, col])
                    out_v[d, pl.ds(j8 * LANES, LANES)] = g
                return c2

            lax.fori_loop(0, D_DIM, d_body, 0)

        def fire_out(l, out_v, osem):
            pltpu.async_copy(
                out_v, out_hbm.at[l].at[:, pl.ds(b0, B_PER_W)], osem)

        def wait_out(l, out_v, osem):
            pltpu.make_async_copy(
                out_v, out_hbm.at[l].at[:, pl.ds(b0, B_PER_W)], osem).wait()

        fire_gather(0, rows_a, gsem_a)
        fire_gather(1, rows_b, gsem_b)
        wait_gather(0, rows_a, gsem_a)
        process(0, rows_a, out_a)
        fire_out(0, out_a, osem_a)
        fire_gather(2, rows_a, gsem_a)
        wait_gather(1, rows_b, gsem_b)
        process(1, rows_b, out_b)
        fire_out(1, out_b, osem_b)
        fire_gather(3, rows_b, gsem_b)

        last = SEQ_L - 1

        def pair_body(p, carry):
            l0 = 2 * p
            wait_gather(l0, rows_a, gsem_a)
            wait_out(l0 - 2, out_a, osem_a)
            process(l0, rows_a, out_a)
            fire_out(l0, out_a, osem_a)
            fire_gather(jnp.minimum(l0 + 2, last), rows_a, gsem_a)
            wait_gather(l0 + 1, rows_b, gsem_b)
            wait_out(l0 - 1, out_b, osem_b)
            process(l0 + 1, rows_b, out_b)
            fire_out(l0 + 1, out_b, osem_b)
            fire_gather(jnp.minimum(l0 + 3, last), rows_b, gsem_b)
            return carry

        lax.fori_loop(1, SEQ_L // 2, pair_body, 0)

        wait_gather(last, rows_a, gsem_a)
        wait_gather(last, rows_b, gsem_b)
        wait_out(SEQ_L - 2, out_a, osem_a)
        wait_out(SEQ_L - 1, out_b, osem_b)

    return body(table2, idx_t, pe)


def kernel(inputs, table):
    table2 = jnp.pad(table, ((0, 0), (0, D_DIM)))
    pe = jnp.asarray(_PE_CONST)
    out = _sc_embed(table2, inputs.T, pe)
    return jnp.transpose(out, (2, 0, 1))


# speedup vs baseline: 1.7344x; 1.4774x over previous
import functools

import numpy as np
import jax
import jax.numpy as jnp
from jax import lax
from jax.experimental import pallas as pl
from jax.experimental.pallas import tpu as pltpu
from jax.experimental.pallas import tpu_sc as plsc

VOCAB_SIZE = 1000000
D_DIM = 64
BATCH_N = 4096
SEQ_L = 200


def _pe_table() -> np.ndarray:
    pos = np.arange(SEQ_L)[:, np.newaxis].astype(np.float64)
    i = np.arange(D_DIM)[np.newaxis, :].astype(np.float64)
    angle_rates = 1.0 / np.power(10000, 2 * (i // 2) / np.float32(D_DIM))
    angle_rads = pos * angle_rates
    angle_rads[:, 0::2] = np.sin(angle_rads[:, 0::2])
    angle_rads[:, 1::2] = np.cos(angle_rads[:, 1::2])
    return np.asarray(angle_rads, dtype=np.float32)


_PE_CONST = _pe_table()

_INFO = plsc.get_sparse_core_info()
_NC, _NS = _INFO.num_cores, _INFO.num_subcores
NW = _NC * _NS

N_ROWS = BATCH_N * SEQ_L
PER_W = N_ROWS // NW
SEQ_PER_CHUNK = 4
CHUNK = SEQ_PER_CHUNK * SEQ_L
NCHUNK = PER_W // CHUNK
_SUBS = [(0, 128), (128, 128), (256, 128), (384, 128),
         (512, 128), (640, 128), (768, 32)]
LANES = 16
VECS_PER_ROW = D_DIM // LANES


def _sc_embed(table, idx_flat, pe):
    mesh = plsc.VectorSubcoreMesh(core_axis_name="c", subcore_axis_name="s")

    @functools.partial(
        pl.kernel,
        mesh=mesh,
        out_type=jax.ShapeDtypeStruct((N_ROWS, D_DIM), jnp.float32),
        scratch_types=[
            pltpu.VMEM((CHUNK,), jnp.int32),
            pltpu.VMEM((CHUNK,), jnp.int32),
            pltpu.VMEM((CHUNK, D_DIM), jnp.float32),
            pltpu.VMEM((CHUNK, D_DIM), jnp.float32),
            pltpu.VMEM((SEQ_L, D_DIM), jnp.float32),
            pltpu.SemaphoreType.DMA,
            pltpu.SemaphoreType.DMA,
            pltpu.SemaphoreType.DMA,
            pltpu.SemaphoreType.DMA,
        ],
        compiler_params=pltpu.CompilerParams(use_tc_tiling_on_sc=False),
    )
    def body(table_hbm, idx_hbm, pe_hbm, out_hbm,
             idx_a, idx_b, rows_a, rows_b, pe_v,
             gsem_a, gsem_b, osem_a, osem_b):
        wid = lax.axis_index("s") * _NC + lax.axis_index("c")
        base = wid * PER_W
        pltpu.sync_copy(pe_hbm, pe_v)

        def fire(g, idx_v, rows_v, gsem):
            row0 = base + g * CHUNK
            pltpu.sync_copy(idx_hbm.at[pl.ds(row0, CHUNK)], idx_v)
            for off, ln in _SUBS:
                pltpu.async_copy(
                    table_hbm.at[idx_v.at[pl.ds(off, ln)]],
                    rows_v.at[pl.ds(off, ln)],
                    gsem,
                )

        def wait_g(idx_v, rows_v, gsem):
            for off, ln in _SUBS:
                pltpu.make_async_copy(
                    table_hbm.at[idx_v.at[pl.ds(off, ln)]],
                    rows_v.at[pl.ds(off, ln)],
                    gsem,
                ).wait()

        def process(rows_v):
            def add_body(r, c2):
                for s in range(SEQ_PER_CHUNK):
                    row = s * SEQ_L + r
                    for j in range(VECS_PER_ROW):
                        sl = pl.ds(j * LANES, LANES)
                        rows_v[row, sl] = rows_v[row, sl] + pe_v[r, sl]
                return c2

            lax.fori_loop(0, SEQ_L, add_body, 0)

        def fire_out(g, rows_v, osem):
            pltpu.async_copy(
                rows_v, out_hbm.at[pl.ds(base + g * CHUNK, CHUNK)], osem)

        def wait_out(g, rows_v, osem):
            pltpu.make_async_copy(
                rows_v, out_hbm.at[pl.ds(base + g * CHUNK, CHUNK)],
                osem).wait()

        last = NCHUNK - 1

        fire(0, idx_a, rows_a, gsem_a)
        fire(1, idx_b, rows_b, gsem_b)
        wait_g(idx_a, rows_a, gsem_a)
        process(rows_a)
        fire_out(0, rows_a, osem_a)
        wait_g(idx_b, rows_b, gsem_b)
        process(rows_b)
        fire_out(1, rows_b, osem_b)
        wait_out(0, rows_a, osem_a)
        fire(2, idx_a, rows_a, gsem_a)
        wait_out(1, rows_b, osem_b)
        fire(3, idx_b, rows_b, gsem_b)

        def pair_body(p, carry):
            g0 = 2 * p
            wait_g(idx_a, rows_a, gsem_a)
            process(rows_a)
            fire_out(g0, rows_a, osem_a)
            wait_g(idx_b, rows_b, gsem_b)
            process(rows_b)
            fire_out(g0 + 1, rows_b, osem_b)
            wait_out(g0, rows_a, osem_a)
            fire(jnp.minimum(g0 + 2, last), idx_a, rows_a, gsem_a)
            wait_out(g0 + 1, rows_b, osem_b)
            fire(jnp.minimum(g0 + 3, last), idx_b, rows_b, gsem_b)
            return carry

        lax.fori_loop(1, NCHUNK // 2, pair_body, 0)

        wait_g(idx_a, rows_a, gsem_a)
        wait_g(idx_b, rows_b, gsem_b)

    return body(table, idx_flat, pe)


def kernel(inputs, table):
    idx_flat = inputs.reshape(-1)
    pe = jnp.asarray(_PE_CONST)
    out = _sc_embed(table, idx_flat, pe)
    return out.reshape(BATCH_N, SEQ_L, D_DIM)
